# baseline (speedup 1.0000x reference)
import functools

import jax
import jax.numpy as jnp
from jax import lax
from jax.experimental import pallas as pl
from jax.experimental.pallas import tpu as pltpu
from jax.experimental.pallas import tpu_sc as plsc

EMB = 64
REPR = 32
NREL = 64


def _sc_gather(idx2d, node_emb, n_rows):
    nw, n_ch, ch = idx2d.shape
    info = plsc.get_sparse_core_info()
    mesh = plsc.VectorSubcoreMesh(core_axis_name="c", subcore_axis_name="s")
    per_w = n_ch * ch

    @functools.partial(
        pl.kernel,
        out_type=jax.ShapeDtypeStruct((n_rows, EMB), jnp.float32),
        mesh=mesh,
        scratch_types=[
            pltpu.VMEM((n_ch, ch), jnp.int32),
            pltpu.VMEM((ch, EMB), jnp.float32),
            pltpu.VMEM((ch, EMB), jnp.float32),
            pltpu.SemaphoreType.DMA,
            pltpu.SemaphoreType.DMA,
        ],
        compiler_params=pltpu.CompilerParams(use_tc_tiling_on_sc=False),
    )
    def k(idx_hbm, table_hbm, out_hbm, idx_v, rows0, rows1, sem0, sem1):
        wid = lax.axis_index("s") * info.num_cores + lax.axis_index("c")
        pltpu.sync_copy(idx_hbm.at[wid], idx_v)
        base = wid * per_w

        pltpu.async_copy(table_hbm.at[idx_v.at[0]], rows0, sem0)

        def body(c2, carry):
            c = c2 * 2
            pltpu.async_copy(table_hbm.at[idx_v.at[c + 1]], rows1, sem1)
            pltpu.make_async_copy(
                table_hbm.at[idx_v.at[c]], rows0, sem0).wait()
            pltpu.sync_copy(rows0, out_hbm.at[pl.ds(base + c * ch, ch)])

            @pl.when(c2 + 1 < n_ch // 2)
            def _():
                pltpu.async_copy(table_hbm.at[idx_v.at[c + 2]], rows0, sem0)

            pltpu.make_async_copy(
                table_hbm.at[idx_v.at[c + 1]], rows1, sem1).wait()
            pltpu.sync_copy(rows1, out_hbm.at[pl.ds(base + (c + 1) * ch, ch)])
            return carry

        lax.fori_loop(0, n_ch // 2, body, 0)

    return k(idx2d, node_emb)


def _tc_loss_sum(x2, r_col, smat, waug, block_e):
    e_total = x2.shape[0]
    nblk = e_total // block_e
    ncol = NREL * REPR

    def body(x_ref, r_ref, s_ref, w_ref, out_ref):
        i = pl.program_id(0)
        xb = x_ref[...].astype(jnp.bfloat16)
        d = jnp.dot(s_ref[...], xb, preferred_element_type=jnp.float32)
        r = r_ref[...]
        oh = (lax.broadcasted_iota(jnp.int32, (block_e, NREL), 1) == r)
        dp = jnp.concatenate(
            [d.astype(jnp.bfloat16), oh.astype(jnp.bfloat16)], axis=1)
        t = jnp.dot(dp, w_ref[...], preferred_element_type=jnp.float32)
        colrel = lax.shift_right_logical(
            lax.broadcasted_iota(jnp.int32, (block_e, ncol), 1), 5)
        sel = jnp.where(colrel == r, t, 0.0)
        s = jnp.sum(sel * sel)

        @pl.when(i == 0)
        def _():
            out_ref[...] = jnp.zeros_like(out_ref)

        out_ref[...] += s

    out = pl.pallas_call(
        body,
        grid=(nblk,),
        in_specs=[
            pl.BlockSpec((2 * block_e, EMB), lambda i: (i, 0)),
            pl.BlockSpec((block_e, 1), lambda i: (i, 0)),
            pl.BlockSpec((block_e, 2 * block_e), lambda i: (0, 0)),
            pl.BlockSpec((2 * EMB, ncol), lambda i: (0, 0)),
        ],
        out_specs=pl.BlockSpec((1, 1), lambda i: (0, 0)),
        out_shape=jax.ShapeDtypeStruct((1, 1), jnp.float32),
    )(x2, r_col, smat, waug)
    return out[0, 0]


def kernel(edge_index_t, edge_attr, node_emb, r_emb_w, r_proj_w):
    e_total = edge_index_t.shape[0]
    n_rows = 2 * e_total

    nw, ch = 32, 128
    n_ch = n_rows // (nw * ch)
    idx2d = edge_index_t.reshape(nw, n_ch, ch)

    x2 = _sc_gather(idx2d, node_emb, n_rows)

    wt = r_proj_w.reshape(NREL, EMB, REPR).transpose(1, 0, 2).reshape(
        EMB, NREL * REPR)
    wtile = jnp.broadcast_to(r_emb_w[:, None, :], (NREL, NREL, REPR)).reshape(
        NREL, NREL * REPR)
    waug = jnp.concatenate([wt, wtile], axis=0).astype(jnp.bfloat16)

    r_col = edge_attr[:, 1:2]

    be = 512
    ir = jnp.arange(be)[:, None]
    ic = jnp.arange(2 * be)[None, :]
    smat = ((ic == 2 * ir).astype(jnp.bfloat16)
            - (ic == 2 * ir + 1).astype(jnp.bfloat16))

    total = _tc_loss_sum(x2, r_col, smat, waug, block_e=be)
    return total / jnp.float32(e_total * REPR)


# speedup vs baseline: 2.1195x; 2.1195x over previous
import functools

import jax
import jax.numpy as jnp
from jax import lax
from jax.experimental import pallas as pl
from jax.experimental.pallas import tpu as pltpu
from jax.experimental.pallas import tpu_sc as plsc

EMB = 64
REPR = 32
NREL = 64


def _sc_gather(idx2d, node_emb, n_rows):
    nw, n_ch, ch = idx2d.shape
    info = plsc.get_sparse_core_info()
    mesh = plsc.VectorSubcoreMesh(core_axis_name="c", subcore_axis_name="s")
    per_w = n_ch * ch

    @functools.partial(
        pl.kernel,
        out_type=jax.ShapeDtypeStruct((n_rows, EMB), jnp.float32),
        mesh=mesh,
        scratch_types=[
            pltpu.VMEM((n_ch, ch), jnp.int32),
            pltpu.VMEM((ch, EMB), jnp.float32),
            pltpu.VMEM((ch, EMB), jnp.float32),
            pltpu.SemaphoreType.DMA,
            pltpu.SemaphoreType.DMA,
        ],
        compiler_params=pltpu.CompilerParams(use_tc_tiling_on_sc=False),
    )
    def k(idx_hbm, table_hbm, out_hbm, idx_v, rows0, rows1, sem0, sem1):
        wid = lax.axis_index("s") * info.num_cores + lax.axis_index("c")
        pltpu.sync_copy(idx_hbm.at[wid], idx_v)
        base = wid * per_w

        pltpu.async_copy(table_hbm.at[idx_v.at[0]], rows0, sem0)

        def body(c2, carry):
            c = c2 * 2
            pltpu.async_copy(table_hbm.at[idx_v.at[c + 1]], rows1, sem1)
            pltpu.make_async_copy(
                table_hbm.at[idx_v.at[c]], rows0, sem0).wait()
            pltpu.sync_copy(rows0, out_hbm.at[pl.ds(base + c * ch, ch)])

            @pl.when(c2 + 1 < n_ch // 2)
            def _():
                pltpu.async_copy(table_hbm.at[idx_v.at[c + 2]], rows0, sem0)

            pltpu.make_async_copy(
                table_hbm.at[idx_v.at[c + 1]], rows1, sem1).wait()
            pltpu.sync_copy(rows1, out_hbm.at[pl.ds(base + (c + 1) * ch, ch)])
            return carry

        lax.fori_loop(0, n_ch // 2, body, 0)

    return k(idx2d, node_emb)


def _tc_loss_sum(x2, r_col, waug, block_e):
    e_total = x2.shape[0]
    nblk = e_total // block_e
    ncol = NREL * REPR

    def body(x_ref, r_ref, w_ref, out_ref):
        i = pl.program_id(0)
        x = x_ref[...]
        d = x[:, :EMB] - x[:, EMB:]
        r = r_ref[...]
        oh = (lax.broadcasted_iota(jnp.int32, (block_e, NREL), 1) == r)
        dp = jnp.concatenate(
            [d.astype(jnp.bfloat16), oh.astype(jnp.bfloat16)], axis=1)
        t = jnp.dot(dp, w_ref[...], preferred_element_type=jnp.float32)
        colrel = lax.shift_right_logical(
            lax.broadcasted_iota(jnp.int32, (block_e, ncol), 1), 5)
        sel = jnp.where(colrel == r, t, 0.0)
        s = jnp.sum(sel * sel)

        @pl.when(i == 0)
        def _():
            out_ref[...] = jnp.zeros_like(out_ref)

        out_ref[...] += s

    out = pl.pallas_call(
        body,
        grid=(nblk,),
        in_specs=[
            pl.BlockSpec((block_e, 2 * EMB), lambda i: (i, 0)),
            pl.BlockSpec((block_e, 1), lambda i: (i, 0)),
            pl.BlockSpec((2 * EMB, ncol), lambda i: (0, 0)),
        ],
        out_specs=pl.BlockSpec((1, 1), lambda i: (0, 0)),
        out_shape=jax.ShapeDtypeStruct((1, 1), jnp.float32),
    )(x2, r_col, waug)
    return out[0, 0]


def kernel(edge_index_t, edge_attr, node_emb, r_emb_w, r_proj_w):
    e_total = edge_index_t.shape[0]
    n_rows = 2 * e_total

    nw, ch = 32, 128
    n_ch = n_rows // (nw * ch)
    idx2d = edge_index_t.reshape(nw, n_ch, ch)

    x = _sc_gather(idx2d, node_emb, n_rows)
    x2 = x.reshape(e_total, 2 * EMB)

    wt = r_proj_w.reshape(NREL, EMB, REPR).transpose(1, 0, 2).reshape(
        EMB, NREL * REPR)
    wtile = jnp.broadcast_to(r_emb_w[:, None, :], (NREL, NREL, REPR)).reshape(
        NREL, NREL * REPR)
    waug = jnp.concatenate([wt, wtile], axis=0).astype(jnp.bfloat16)

    r_col = edge_attr[:, 1:2]

    total = _tc_loss_sum(x2, r_col, waug, block_e=512)
    return total / jnp.float32(e_total * REPR)
